# baseline (speedup 1.0000x reference)
import functools

import jax
import jax.numpy as jnp
from jax import lax
from jax.experimental import pallas as pl
from jax.experimental.pallas import tpu as pltpu
from jax.experimental.pallas import tpu_sc as plsc

_CHUNK = 128
_DP = 128


def _sc_gather(table, idx3d, n_out, nbuf):
    rows_w = idx3d.shape[1]

    mesh = plsc.VectorSubcoreMesh(core_axis_name="c", subcore_axis_name="s")

    @functools.partial(
        pl.kernel,
        out_type=jax.ShapeDtypeStruct((n_out, _DP), jnp.bfloat16),
        mesh=mesh,
        compiler_params=pltpu.CompilerParams(use_tc_tiling_on_sc=False),
        scratch_types=[
            pltpu.VMEM((rows_w, _CHUNK), jnp.int32),
            pltpu.VMEM((nbuf, _CHUNK, _DP), jnp.bfloat16),
            pltpu.SemaphoreType.DMA,
            pltpu.SemaphoreType.DMA,
        ],
    )
    def sc_kernel(tab_hbm, idx_hbm, out_hbm, idx_v, rows_v, gsem, wsem):
        wid = lax.axis_index("s") * 2 + lax.axis_index("c")
        pltpu.sync_copy(idx_hbm.at[wid], idx_v)
        base = wid * rows_w * _CHUNK

        @pl.loop(0, rows_w, step=nbuf)
        def _(c0):
            gathers = [
                pltpu.async_copy(tab_hbm.at[idx_v.at[c0 + b]], rows_v.at[b],
                                 gsem)
                for b in range(nbuf)
            ]
            for cp in gathers:
                cp.wait()
            writes = [
                pltpu.async_copy(
                    rows_v.at[b],
                    out_hbm.at[pl.ds(base + (c0 + b) * _CHUNK, _CHUNK)],
                    wsem)
                for b in range(nbuf)
            ]
            for cp in writes:
                cp.wait()

    return sc_kernel(table, idx3d)


def _tc_scores(art_rows, lab_rows, batch, k, bb):
    nb = batch // bb
    rows_o = bb // 128

    def body(a_ref, g_ref, dots_ref):
        a = a_ref[...]
        g = g_ref[...]
        p = a * g
        ones_row = jnp.ones((1, _DP), dtype=jnp.bfloat16)
        s = jax.lax.dot_general(ones_row, p, (((1,), (1,)), ((), ())),
                                preferred_element_type=jnp.float32)
        dots_ref[...] = s.reshape(rows_o, 128)

    dots = pl.pallas_call(
        body,
        grid=(nb, k),
        in_specs=[
            pl.BlockSpec((bb, _DP), lambda i, j: (i, 0)),
            pl.BlockSpec((bb, _DP), lambda i, j: (j * nb + i, 0)),
        ],
        out_specs=pl.BlockSpec((rows_o, 128), lambda i, j: (j * nb + i, 0)),
        out_shape=jax.ShapeDtypeStruct((k * batch // 128, 128), jnp.float32),
    )(art_rows, lab_rows)

    def norm_body(a_ref, g_ref, norm_ref):
        @pl.when(pl.program_id(0) == 0)
        def _():
            norm_ref[...] = jnp.zeros_like(norm_ref)

        a = a_ref[...].astype(jnp.float32)
        g = g_ref[...].astype(jnp.float32)
        norm_ref[...] += jnp.reshape(jnp.sum(a * a) + jnp.sum(g * g), (1, 1))

    norm = pl.pallas_call(
        norm_body,
        grid=(nb,),
        in_specs=[
            pl.BlockSpec((bb, _DP), lambda i: (i, 0)),
            pl.BlockSpec((bb, _DP), lambda i: (i, 0)),
        ],
        out_specs=pl.BlockSpec((1, 1), lambda i: (0, 0)),
        out_shape=jax.ShapeDtypeStruct((1, 1), jnp.float32),
    )(art_rows, lab_rows)
    return dots, norm


def kernel(art_embed, lab_embed, artist_idx, label_idx, noise_idxs):
    batch = artist_idx.shape[0]
    d = art_embed.shape[1]
    n_neg = noise_idxs.shape[1]
    k = n_neg + 1

    lab_p = jnp.pad(lab_embed.astype(jnp.bfloat16), ((0, 0), (0, _DP - d)))
    art_p = jnp.pad(art_embed.astype(jnp.bfloat16), ((0, 0), (0, _DP - d)))

    lab_all = jnp.concatenate(
        [label_idx[None, :], noise_idxs.T.astype(jnp.int32)], axis=0)
    lab_idx3d = lab_all.reshape(32, -1, _CHUNK)
    art_idx3d = artist_idx.astype(jnp.int32).reshape(32, -1, _CHUNK)

    lab_rows = _sc_gather(lab_p, lab_idx3d, k * batch, nbuf=6)
    art_rows = _sc_gather(art_p, art_idx3d, batch, nbuf=4)

    dots, norm = _tc_scores(art_rows, lab_rows, batch, k, bb=2048)

    dots2 = dots.reshape(k, batch)
    scores = dots2[0][:, None]
    noise_scores = dots2[1:].T
    embed_norm = norm[0, 0] / jnp.float32(batch * d)
    return scores, noise_scores, embed_norm


# speedup vs baseline: 2.2252x; 2.2252x over previous
import functools

import jax
import jax.numpy as jnp
from jax import lax
from jax.experimental import pallas as pl
from jax.experimental.pallas import tpu as pltpu
from jax.experimental.pallas import tpu_sc as plsc

_CHUNK = 128
_DP = 128


def _sc_gather(table, idx3d, n_out, nbuf):
    rows_w = idx3d.shape[1]

    mesh = plsc.VectorSubcoreMesh(core_axis_name="c", subcore_axis_name="s")

    @functools.partial(
        pl.kernel,
        out_type=jax.ShapeDtypeStruct((n_out, _DP), jnp.float32),
        mesh=mesh,
        compiler_params=pltpu.CompilerParams(use_tc_tiling_on_sc=False),
        scratch_types=[
            pltpu.VMEM((rows_w, _CHUNK), jnp.int32),
            pltpu.VMEM((nbuf, _CHUNK, _DP), jnp.float32),
            pltpu.SemaphoreType.DMA,
            pltpu.SemaphoreType.DMA,
        ],
    )
    def sc_kernel(tab_hbm, idx_hbm, out_hbm, idx_v, rows_v, gsem, wsem):
        wid = lax.axis_index("s") * 2 + lax.axis_index("c")
        pltpu.sync_copy(idx_hbm.at[wid], idx_v)
        base = wid * rows_w * _CHUNK

        @pl.loop(0, rows_w, step=nbuf)
        def _(c0):
            gathers = [
                pltpu.async_copy(tab_hbm.at[idx_v.at[c0 + b]], rows_v.at[b],
                                 gsem)
                for b in range(nbuf)
            ]
            for cp in gathers:
                cp.wait()
            writes = [
                pltpu.async_copy(
                    rows_v.at[b],
                    out_hbm.at[pl.ds(base + (c0 + b) * _CHUNK, _CHUNK)],
                    wsem)
                for b in range(nbuf)
            ]
            for cp in writes:
                cp.wait()

    return sc_kernel(table, idx3d)


def _tc_scores(art_rows, lab_rows, batch, k, bb):
    nb = batch // bb
    rows_o = bb // 128

    def body(a_ref, g_ref, dots_ref):
        a = a_ref[...]
        g = g_ref[...]
        p = a * g
        ones_row = jnp.ones((1, _DP), dtype=jnp.float32)
        s = jax.lax.dot_general(ones_row, p, (((1,), (1,)), ((), ())),
                                preferred_element_type=jnp.float32)
        dots_ref[...] = s.reshape(rows_o, 128)

    dots = pl.pallas_call(
        body,
        grid=(nb, k),
        in_specs=[
            pl.BlockSpec((bb, _DP), lambda i, j: (i, 0)),
            pl.BlockSpec((bb, _DP), lambda i, j: (j * nb + i, 0)),
        ],
        out_specs=pl.BlockSpec((rows_o, 128), lambda i, j: (j * nb + i, 0)),
        out_shape=jax.ShapeDtypeStruct((k * batch // 128, 128), jnp.float32),
    )(art_rows, lab_rows)

    def norm_body(a_ref, g_ref, norm_ref):
        @pl.when(pl.program_id(0) == 0)
        def _():
            norm_ref[...] = jnp.zeros_like(norm_ref)

        a = a_ref[...]
        g = g_ref[...]
        norm_ref[...] += jnp.reshape(jnp.sum(a * a) + jnp.sum(g * g), (1, 1))

    norm = pl.pallas_call(
        norm_body,
        grid=(nb,),
        in_specs=[
            pl.BlockSpec((bb, _DP), lambda i: (i, 0)),
            pl.BlockSpec((bb, _DP), lambda i: (i, 0)),
        ],
        out_specs=pl.BlockSpec((1, 1), lambda i: (0, 0)),
        out_shape=jax.ShapeDtypeStruct((1, 1), jnp.float32),
    )(art_rows, lab_rows)
    return dots, norm


def kernel(art_embed, lab_embed, artist_idx, label_idx, noise_idxs):
    batch = artist_idx.shape[0]
    d = art_embed.shape[1]
    n_neg = noise_idxs.shape[1]
    k = n_neg + 1

    lab_p = jnp.pad(lab_embed, ((0, 0), (0, _DP - d)))
    art_p = jnp.pad(art_embed, ((0, 0), (0, _DP - d)))

    lab_all = jnp.concatenate(
        [label_idx[None, :], noise_idxs.T.astype(jnp.int32)], axis=0)
    lab_idx3d = lab_all.reshape(32, -1, _CHUNK)
    art_idx3d = artist_idx.astype(jnp.int32).reshape(32, -1, _CHUNK)

    lab_rows = _sc_gather(lab_p, lab_idx3d, k * batch, nbuf=6)
    art_rows = _sc_gather(art_p, art_idx3d, batch, nbuf=4)

    dots, norm = _tc_scores(art_rows, lab_rows, batch, k, bb=2048)

    dots2 = dots.reshape(k, batch)
    scores = dots2[0][:, None]
    noise_scores = dots2[1:].T
    embed_norm = norm[0, 0] / jnp.float32(batch * d)
    return scores, noise_scores, embed_norm


# speedup vs baseline: 2.2327x; 1.0034x over previous
import functools

import jax
import jax.numpy as jnp
from jax import lax
from jax.experimental import pallas as pl
from jax.experimental.pallas import tpu as pltpu
from jax.experimental.pallas import tpu_sc as plsc

_CHUNK = 128
_DP = 128


def _sc_gather(table, idx3d, n_out, nbuf):
    rows_w = idx3d.shape[1]

    mesh = plsc.VectorSubcoreMesh(core_axis_name="c", subcore_axis_name="s")

    @functools.partial(
        pl.kernel,
        out_type=jax.ShapeDtypeStruct((n_out, _DP), jnp.float32),
        mesh=mesh,
        compiler_params=pltpu.CompilerParams(use_tc_tiling_on_sc=False),
        scratch_types=[
            pltpu.VMEM((rows_w, _CHUNK), jnp.int32),
            pltpu.VMEM((nbuf, _CHUNK, _DP), jnp.float32),
            pltpu.SemaphoreType.DMA,
            pltpu.SemaphoreType.DMA,
        ],
    )
    def sc_kernel(tab_hbm, idx_hbm, out_hbm, idx_v, rows_v, gsem, wsem):
        wid = lax.axis_index("s") * 2 + lax.axis_index("c")
        pltpu.sync_copy(idx_hbm.at[wid], idx_v)
        base = wid * rows_w * _CHUNK

        @pl.loop(0, rows_w, step=nbuf)
        def _(c0):
            gathers = [
                pltpu.async_copy(tab_hbm.at[idx_v.at[c0 + b]], rows_v.at[b],
                                 gsem)
                for b in range(nbuf)
            ]
            for cp in gathers:
                cp.wait()
            writes = [
                pltpu.async_copy(
                    rows_v.at[b],
                    out_hbm.at[pl.ds(base + (c0 + b) * _CHUNK, _CHUNK)],
                    wsem)
                for b in range(nbuf)
            ]
            for cp in writes:
                cp.wait()

    return sc_kernel(table, idx3d)


def _tc_scores(art_rows, lab_rows, batch, k, bb):
    nb = batch // bb
    rows_o = bb // 128

    def body(a_ref, g_ref, dots_ref, norm_ref):
        i = pl.program_id(0)
        j = pl.program_id(1)
        a = a_ref[...]
        g = g_ref[...]
        p = a * g
        ones_row = jnp.ones((1, _DP), dtype=jnp.float32)
        s = jax.lax.dot_general(ones_row, p, (((1,), (1,)), ((), ())),
                                preferred_element_type=jnp.float32)
        dots_ref[...] = s.reshape(rows_o, 128)

        @pl.when(jnp.logical_and(i == 0, j == 0))
        def _():
            norm_ref[...] = jnp.zeros_like(norm_ref)

        @pl.when(j == 0)
        def _():
            norm_ref[...] += jnp.reshape(jnp.sum(a * a) + jnp.sum(g * g),
                                         (1, 1))

    dots, norm = pl.pallas_call(
        body,
        grid=(nb, k),
        in_specs=[
            pl.BlockSpec((bb, _DP), lambda i, j: (i, 0)),
            pl.BlockSpec((bb, _DP), lambda i, j: (j * nb + i, 0)),
        ],
        out_specs=[
            pl.BlockSpec((rows_o, 128), lambda i, j: (j * nb + i, 0)),
            pl.BlockSpec((1, 1), lambda i, j: (0, 0)),
        ],
        out_shape=[
            jax.ShapeDtypeStruct((k * batch // 128, 128), jnp.float32),
            jax.ShapeDtypeStruct((1, 1), jnp.float32),
        ],
    )(art_rows, lab_rows)
    return dots, norm


def kernel(art_embed, lab_embed, artist_idx, label_idx, noise_idxs):
    batch = artist_idx.shape[0]
    d = art_embed.shape[1]
    n_neg = noise_idxs.shape[1]
    k = n_neg + 1

    lab_p = jnp.pad(lab_embed, ((0, 0), (0, _DP - d)))
    art_p = jnp.pad(art_embed, ((0, 0), (0, _DP - d)))

    lab_all = jnp.concatenate(
        [label_idx[None, :], noise_idxs.T.astype(jnp.int32)], axis=0)
    lab_idx3d = lab_all.reshape(32, -1, _CHUNK)
    art_idx3d = artist_idx.astype(jnp.int32).reshape(32, -1, _CHUNK)

    lab_rows = _sc_gather(lab_p, lab_idx3d, k * batch, nbuf=6)
    art_rows = _sc_gather(art_p, art_idx3d, batch, nbuf=4)

    dots, norm = _tc_scores(art_rows, lab_rows, batch, k, bb=2048)

    dots2 = dots.reshape(k, batch)
    scores = dots2[0][:, None]
    noise_scores = dots2[1:].T
    embed_norm = norm[0, 0] / jnp.float32(batch * d)
    return scores, noise_scores, embed_norm
